# baseline (speedup 1.0000x reference)
import jax
import jax.numpy as jnp
from jax.experimental import pallas as pl
from jax.experimental.pallas import tpu as pltpu

D_MODEL = 1024
TIME_STEPS = 8
BATCH = 2
SEQ = 2048

NS = 2
CH = SEQ // NS


def _encode_kernel(w_ref, emb_ref, rand_ref, out_ref, rate_s, st_s):
    t = pl.program_id(2)

    a = w_ref[0]
    b = w_ref[1]
    m = jnp.maximum(a, b)
    e0 = jnp.exp(a - m)
    e1 = jnp.exp(b - m)
    denom = e0 + e1
    w0 = e0 / denom
    w1 = e1 / denom

    @pl.when(t == 0)
    def _():
        r = jax.nn.sigmoid(emb_ref[0])
        rate_s[...] = r
        st_s[...] = (r * (TIME_STEPS - 1)).astype(jnp.int32)

    rate = rate_s[...]
    st = st_s[...]
    out_ref[...] = jnp.where(st == t, w1, 0.0) + jnp.where(
        rand_ref[...] < rate, w0, 0.0
    )


@jax.jit
def kernel(embeddings, encoding_weights, random_vals):
    rand2d = random_vals.reshape(BATCH * TIME_STEPS * SEQ, D_MODEL)
    grid = (BATCH, NS, TIME_STEPS)
    out = pl.pallas_call(
        _encode_kernel,
        grid=grid,
        in_specs=[
            pl.BlockSpec(memory_space=pltpu.SMEM),
            pl.BlockSpec((1, CH, D_MODEL), lambda b, sh, t: (b, sh, 0)),
            pl.BlockSpec(
                (CH, D_MODEL),
                lambda b, sh, t: (b * TIME_STEPS * NS + t * NS + sh, 0),
            ),
        ],
        out_specs=pl.BlockSpec(
            (CH, D_MODEL),
            lambda b, sh, t: (b * TIME_STEPS * NS + t * NS + sh, 0),
        ),
        out_shape=jax.ShapeDtypeStruct(
            (BATCH * TIME_STEPS * SEQ, D_MODEL), jnp.float32
        ),
        scratch_shapes=[
            pltpu.VMEM((CH, D_MODEL), jnp.float32),
            pltpu.VMEM((CH, D_MODEL), jnp.int32),
        ],
        compiler_params=pltpu.CompilerParams(
            dimension_semantics=("arbitrary", "arbitrary", "arbitrary"),
        ),
    )(encoding_weights, embeddings, rand2d)
    return out.reshape(BATCH, TIME_STEPS, SEQ, D_MODEL)


# speedup vs baseline: 1.0713x; 1.0713x over previous
import jax
import jax.numpy as jnp
from jax.experimental import pallas as pl
from jax.experimental.pallas import tpu as pltpu

D_MODEL = 1024
TIME_STEPS = 8
BATCH = 2
SEQ = 2048

BS = 256


def _encode_kernel(w_ref, emb_ref, rand_ref, out_ref):
    a = w_ref[0]
    b = w_ref[1]
    m = jnp.maximum(a, b)
    e0 = jnp.exp(a - m)
    e1 = jnp.exp(b - m)
    denom = e0 + e1
    w0 = e0 / denom
    w1 = e1 / denom

    rate = jax.nn.sigmoid(emb_ref[0])
    spike_time = (rate * (TIME_STEPS - 1)).astype(jnp.int32)
    for t in range(TIME_STEPS):
        lo = jnp.where(spike_time == t, w1, 0.0)
        out_ref[0, t] = lo + jnp.where(rand_ref[0, t] < rate, w0, 0.0)


@jax.jit
def kernel(embeddings, encoding_weights, random_vals):
    grid = (BATCH, SEQ // BS)
    return pl.pallas_call(
        _encode_kernel,
        grid=grid,
        in_specs=[
            pl.BlockSpec(memory_space=pltpu.SMEM),
            pl.BlockSpec((1, BS, D_MODEL), lambda b, s: (b, s, 0)),
            pl.BlockSpec((1, TIME_STEPS, BS, D_MODEL), lambda b, s: (b, 0, s, 0)),
        ],
        out_specs=pl.BlockSpec((1, TIME_STEPS, BS, D_MODEL), lambda b, s: (b, 0, s, 0)),
        out_shape=jax.ShapeDtypeStruct(
            (BATCH, TIME_STEPS, SEQ, D_MODEL), jnp.float32
        ),
        compiler_params=pltpu.CompilerParams(
            dimension_semantics=("parallel", "parallel"),
        ),
    )(encoding_weights, embeddings, random_vals)
